# baseline (speedup 1.0000x reference)
import functools

import jax
import jax.numpy as jnp
from jax import lax
from jax.experimental import pallas as pl
from jax.experimental.pallas import tpu as pltpu
from jax.experimental.pallas import tpu_sc as plsc


def _vq_body(flat_ref, e_ref, idx_ref):
    flat = flat_ref[...]
    e = e_ref[...]
    fsq = jnp.sum(flat * flat, axis=1, keepdims=True)
    esq = jnp.sum(e * e, axis=1)
    m = lax.dot_general(flat, e, (((1,), (1,)), ((), ())),
                        preferred_element_type=jnp.float32)
    d = fsq + esq[None, :] - 2.0 * m
    mn = jnp.min(d, axis=1, keepdims=True)
    ks = lax.broadcasted_iota(jnp.int32, d.shape, 1)
    idx = jnp.min(jnp.where(d == mn, ks, jnp.int32(2 ** 30)), axis=1)
    idx_ref[...] = idx[:, None]


def _vq_argmin(flat, e):
    return pl.pallas_call(
        _vq_body,
        out_shape=jax.ShapeDtypeStruct((flat.shape[0], 1), jnp.int32),
    )(flat, e)


def _make_sc_gather(V, D, B):
    info = plsc.get_sparse_core_info()
    num_cores = info.num_cores
    b_per_w = B // (num_cores * info.num_subcores)
    mesh = plsc.VectorSubcoreMesh(core_axis_name="c", subcore_axis_name="s")

    @functools.partial(
        pl.kernel, mesh=mesh,
        out_type=jax.ShapeDtypeStruct((B, D), jnp.float32),
        scratch_types=[
            pltpu.VMEM((b_per_w,), jnp.int32),
            pltpu.VMEM((b_per_w, D), jnp.float32),
            pltpu.SemaphoreType.DMA,
        ],
    )
    def gather(table_hbm, idx_hbm, out_hbm, idx_v, rows_v, sem):
        wid = lax.axis_index("s") * num_cores + lax.axis_index("c")
        base = wid * b_per_w
        pltpu.sync_copy(idx_hbm.at[pl.ds(base, b_per_w)], idx_v)
        pltpu.async_copy(table_hbm.at[idx_v], rows_v, sem).wait()
        pltpu.sync_copy(rows_v, out_hbm.at[pl.ds(base, b_per_w)])

    return gather


def _stats_body(flat_ref, q_ref, idx_ref, loss_ref, ppx_ref, qst_ref):
    flat = flat_ref[...]
    q = q_ref[...]
    diff = q - flat
    v = jnp.mean(diff * diff)
    loss_ref[...] = jnp.reshape(v + 0.25 * v, (1, 1))
    n = flat.shape[0]
    idx = idx_ref[...]
    eq = (idx == idx.reshape(1, n)).astype(jnp.float32)
    avg = jnp.sum(eq, axis=1) * (1.0 / n)
    ent = jnp.mean(jnp.log(avg + 1e-10))
    ppx_ref[...] = jnp.reshape(jnp.exp(-ent), (1, 1))
    qst_ref[...] = flat + (q - flat)


def _stats(flat, q, idx2):
    n, d = flat.shape
    return pl.pallas_call(
        _stats_body,
        out_shape=[jax.ShapeDtypeStruct((1, 1), jnp.float32),
                   jax.ShapeDtypeStruct((1, 1), jnp.float32),
                   jax.ShapeDtypeStruct((n, d), jnp.float32)],
    )(flat, q, idx2)


def _prep_w(w):
    wt = jnp.transpose(w, (2, 3, 0, 1))
    rows = []
    for a in range(2):
        cols = []
        for b in range(2):
            t1 = []
            for dy in range(2):
                t2 = [wt[3 - a - 2 * dy, 3 - b - 2 * dx] for dx in range(2)]
                t1.append(jnp.stack(t2))
            cols.append(jnp.stack(t1))
        rows.append(jnp.stack(cols))
    return jnp.stack(rows)


def _conv_body(H, W, cin, cout, x_ref, w_ref, b_ref, o_ref):
    a = pl.program_id(1)
    b = pl.program_id(2)
    bias = b_ref[...]
    for aa in range(2):
        for bb in range(2):
            @pl.when(jnp.logical_and(a == aa, b == bb))
            def _(aa=aa, bb=bb):
                acc = None
                for dy in range(2):
                    for dx in range(2):
                        xw = x_ref[0, aa + dy:aa + dy + H,
                                   bb + dx:bb + dx + W, :]
                        p = jnp.dot(xw.reshape(H * W, cin),
                                    w_ref[0, 0, dy, dx],
                                    preferred_element_type=jnp.float32)
                        acc = p if acc is None else acc + p
                o_ref[0, 0, 0] = jnp.maximum(acc + bias, 0.0)


def _deconv_relu_phases(x, wr, bias):
    n, h2, w2, cin = x.shape
    H, W = h2 - 2, w2 - 2
    cout = wr.shape[-1]
    return pl.pallas_call(
        functools.partial(_conv_body, H, W, cin, cout),
        grid=(n, 2, 2),
        in_specs=[
            pl.BlockSpec((1, H + 2, W + 2, cin), lambda i, a, b: (i, 0, 0, 0)),
            pl.BlockSpec((1, 1, 2, 2, cin, cout),
                         lambda i, a, b: (a, b, 0, 0, 0, 0)),
            pl.BlockSpec((1, cout), lambda i, a, b: (0, 0)),
        ],
        out_specs=pl.BlockSpec((1, 1, 1, H * W, cout),
                               lambda i, a, b: (i, a, b, 0, 0)),
        out_shape=jax.ShapeDtypeStruct((n, 2, 2, H * W, cout), jnp.float32),
    )(x, wr, bias.reshape(1, cout))


def _deconv_relu(x, wr, bias):
    n, h2, w2, cin = x.shape
    H, W = h2 - 2, w2 - 2
    cout = wr.shape[-1]
    out = _deconv_relu_phases(x, wr, bias)
    y = out.reshape(n, 2, 2, H, W, cout)
    y = jnp.transpose(y, (0, 3, 1, 4, 2, 5)).reshape(n, 2 * H, 2 * W, cout)
    return y


def _pool_body(H, W, x_ref, o_ref):
    x = x_ref[0]
    m = x[0:H, 0:W, :]
    for dy in range(3):
        for dx in range(3):
            if dy == 0 and dx == 0:
                continue
            m = jnp.maximum(m, x[dy:dy + H, dx:dx + W, :])
    o_ref[0] = m


def _pool_body_cf(H, W, x_ref, o_ref):
    x = x_ref[0]
    m = x[:, 0:H, 0:W]
    for dy in range(3):
        for dx in range(3):
            if dy == 0 and dx == 0:
                continue
            m = jnp.maximum(m, x[:, dy:dy + H, dx:dx + W])
    o_ref[0] = m


def _maxpool3_nchw(x):
    n, c, H, W = x.shape
    xp = jnp.pad(x, ((0, 0), (0, 0), (1, 1), (1, 1)),
                 constant_values=-jnp.inf)
    return pl.pallas_call(
        functools.partial(_pool_body_cf, H, W),
        grid=(n,),
        in_specs=[pl.BlockSpec((1, c, H + 2, W + 2), lambda i: (i, 0, 0, 0))],
        out_specs=pl.BlockSpec((1, c, H, W), lambda i: (i, 0, 0, 0)),
        out_shape=jax.ShapeDtypeStruct((n, c, H, W), jnp.float32),
    )(xp)


def _maxpool3(x):
    n, H, W, c = x.shape
    xp = jnp.pad(x, ((0, 0), (1, 1), (1, 1), (0, 0)),
                 constant_values=-jnp.inf)
    return pl.pallas_call(
        functools.partial(_pool_body, H, W),
        grid=(n,),
        in_specs=[pl.BlockSpec((1, H + 2, W + 2, c), lambda i: (i, 0, 0, 0))],
        out_specs=pl.BlockSpec((1, H, W, c), lambda i: (i, 0, 0, 0)),
        out_shape=jax.ShapeDtypeStruct((n, H, W, c), jnp.float32),
    )(xp)


def kernel(x, E, W1, b1, W2, b2, W3, b3, W4, b4, W5, b5):
    n, cz, hz, wz = x.shape
    V, D = E.shape
    flat = jnp.transpose(x, (0, 2, 3, 1)).reshape(-1, D)

    idx2 = _vq_argmin(flat, E)
    q = _make_sc_gather(V, D, flat.shape[0])(E, idx2.reshape(-1))
    loss, ppx, qst = _stats(flat, q, idx2)

    h = qst.reshape(n, hz, wz, cz)
    for w, b in ((W1, b1), (W2, b2), (W3, b3), (W4, b4)):
        hp = jnp.pad(h, ((0, 0), (1, 1), (1, 1), (0, 0)))
        h = _maxpool3(_deconv_relu(hp, _prep_w(w), b))

    hp = jnp.pad(h, ((0, 0), (1, 1), (1, 1), (0, 0)))
    H = hp.shape[1] - 2
    Wd = hp.shape[2] - 2
    ph = _deconv_relu_phases(hp, _prep_w(W5), b5)
    y = ph.reshape(n, 2, 2, H, Wd, 3)
    y = jnp.transpose(y, (0, 5, 3, 1, 4, 2)).reshape(n, 3, 2 * H, 2 * Wd)
    h = _maxpool3_nchw(y)

    return (loss[0, 0], h, ppx[0, 0])


# speedup vs baseline: 1.0353x; 1.0353x over previous
import functools

import jax
import jax.numpy as jnp
from jax import lax
from jax.experimental import pallas as pl
from jax.experimental.pallas import tpu as pltpu
from jax.experimental.pallas import tpu_sc as plsc


def _vq_body(flat_ref, e_ref, idx_ref):
    flat = flat_ref[...]
    e = e_ref[...]
    fsq = jnp.sum(flat * flat, axis=1, keepdims=True)
    esq = jnp.sum(e * e, axis=1)
    m = lax.dot_general(flat, e, (((1,), (1,)), ((), ())),
                        preferred_element_type=jnp.float32)
    d = fsq + esq[None, :] - 2.0 * m
    mn = jnp.min(d, axis=1, keepdims=True)
    ks = lax.broadcasted_iota(jnp.int32, d.shape, 1)
    idx = jnp.min(jnp.where(d == mn, ks, jnp.int32(2 ** 30)), axis=1)
    idx_ref[...] = idx[:, None]


def _vq_argmin(flat, e):
    return pl.pallas_call(
        _vq_body,
        out_shape=jax.ShapeDtypeStruct((flat.shape[0], 1), jnp.int32),
    )(flat, e)


def _make_sc_gather(V, D, B):
    info = plsc.get_sparse_core_info()
    num_cores = info.num_cores
    b_per_w = B // (num_cores * info.num_subcores)
    mesh = plsc.VectorSubcoreMesh(core_axis_name="c", subcore_axis_name="s")

    @functools.partial(
        pl.kernel, mesh=mesh,
        out_type=jax.ShapeDtypeStruct((B, D), jnp.float32),
        scratch_types=[
            pltpu.VMEM((b_per_w,), jnp.int32),
            pltpu.VMEM((b_per_w, D), jnp.float32),
            pltpu.SemaphoreType.DMA,
        ],
    )
    def gather(table_hbm, idx_hbm, out_hbm, idx_v, rows_v, sem):
        wid = lax.axis_index("s") * num_cores + lax.axis_index("c")
        base = wid * b_per_w
        pltpu.sync_copy(idx_hbm.at[pl.ds(base, b_per_w)], idx_v)
        pltpu.async_copy(table_hbm.at[idx_v], rows_v, sem).wait()
        pltpu.sync_copy(rows_v, out_hbm.at[pl.ds(base, b_per_w)])

    return gather


def _stats_body(flat_ref, q_ref, idx_ref, loss_ref, ppx_ref, qst_ref):
    flat = flat_ref[...]
    q = q_ref[...]
    diff = q - flat
    v = jnp.mean(diff * diff)
    loss_ref[...] = jnp.reshape(v + 0.25 * v, (1, 1))
    n = flat.shape[0]
    idx = idx_ref[...]
    eq = (idx == idx.reshape(1, n)).astype(jnp.float32)
    avg = jnp.sum(eq, axis=1) * (1.0 / n)
    ent = jnp.mean(jnp.log(avg + 1e-10))
    ppx_ref[...] = jnp.reshape(jnp.exp(-ent), (1, 1))
    qst_ref[...] = flat + (q - flat)


def _stats(flat, q, idx2):
    n, d = flat.shape
    return pl.pallas_call(
        _stats_body,
        out_shape=[jax.ShapeDtypeStruct((1, 1), jnp.float32),
                   jax.ShapeDtypeStruct((1, 1), jnp.float32),
                   jax.ShapeDtypeStruct((n, d), jnp.float32)],
    )(flat, q, idx2)


def _prep_w(w):
    wt = jnp.transpose(w, (2, 3, 0, 1))
    rows = []
    for a in range(2):
        cols = []
        for b in range(2):
            t1 = []
            for dy in range(2):
                t2 = [wt[3 - a - 2 * dy, 3 - b - 2 * dx] for dx in range(2)]
                t1.append(jnp.stack(t2))
            cols.append(jnp.stack(t1))
        rows.append(jnp.stack(cols))
    return jnp.stack(rows)


def _conv_body(H, W, cin, cout, x_ref, w_ref, b_ref, o_ref):
    a = pl.program_id(1)
    b = pl.program_id(2)
    bias = b_ref[...]
    for aa in range(2):
        for bb in range(2):
            @pl.when(jnp.logical_and(a == aa, b == bb))
            def _(aa=aa, bb=bb):
                acc = None
                for dy in range(2):
                    for dx in range(2):
                        xw = x_ref[0, aa + dy:aa + dy + H,
                                   bb + dx:bb + dx + W, :]
                        p = jnp.dot(xw.reshape(H * W, cin),
                                    w_ref[0, 0, dy, dx],
                                    preferred_element_type=jnp.float32)
                        acc = p if acc is None else acc + p
                o_ref[0, 0, 0] = jnp.maximum(acc + bias, 0.0)


def _conv_body_nb(N, H, W, cin, cout, x_ref, w_ref, b_ref, o_ref):
    a = pl.program_id(0)
    b = pl.program_id(1)
    bias = b_ref[...]
    for aa in range(2):
        for bb in range(2):
            @pl.when(jnp.logical_and(a == aa, b == bb))
            def _(aa=aa, bb=bb):
                acc = None
                for dy in range(2):
                    for dx in range(2):
                        xw = x_ref[:, aa + dy:aa + dy + H,
                                   bb + dx:bb + dx + W, :]
                        p = jnp.dot(xw.reshape(N * H * W, cin),
                                    w_ref[0, 0, dy, dx],
                                    preferred_element_type=jnp.float32)
                        acc = p if acc is None else acc + p
                o_ref[0, 0] = jnp.maximum(acc + bias, 0.0)


def _deconv_relu_phases_nb(x, wr, bias):
    n, h2, w2, cin = x.shape
    H, W = h2 - 2, w2 - 2
    cout = wr.shape[-1]
    return pl.pallas_call(
        functools.partial(_conv_body_nb, n, H, W, cin, cout),
        grid=(2, 2),
        in_specs=[
            pl.BlockSpec((n, H + 2, W + 2, cin), lambda a, b: (0, 0, 0, 0)),
            pl.BlockSpec((1, 1, 2, 2, cin, cout),
                         lambda a, b: (a, b, 0, 0, 0, 0)),
            pl.BlockSpec((1, cout), lambda a, b: (0, 0)),
        ],
        out_specs=pl.BlockSpec((1, 1, n * H * W, cout),
                               lambda a, b: (a, b, 0, 0)),
        out_shape=jax.ShapeDtypeStruct((2, 2, n * H * W, cout), jnp.float32),
    )(x, wr, bias.reshape(1, cout))


def _deconv_relu_phases(x, wr, bias):
    n, h2, w2, cin = x.shape
    H, W = h2 - 2, w2 - 2
    cout = wr.shape[-1]
    return pl.pallas_call(
        functools.partial(_conv_body, H, W, cin, cout),
        grid=(n, 2, 2),
        in_specs=[
            pl.BlockSpec((1, H + 2, W + 2, cin), lambda i, a, b: (i, 0, 0, 0)),
            pl.BlockSpec((1, 1, 2, 2, cin, cout),
                         lambda i, a, b: (a, b, 0, 0, 0, 0)),
            pl.BlockSpec((1, cout), lambda i, a, b: (0, 0)),
        ],
        out_specs=pl.BlockSpec((1, 1, 1, H * W, cout),
                               lambda i, a, b: (i, a, b, 0, 0)),
        out_shape=jax.ShapeDtypeStruct((n, 2, 2, H * W, cout), jnp.float32),
    )(x, wr, bias.reshape(1, cout))


def _deconv_relu(x, wr, bias):
    n, h2, w2, cin = x.shape
    H, W = h2 - 2, w2 - 2
    cout = wr.shape[-1]
    out = _deconv_relu_phases_nb(x, wr, bias)
    y = out.reshape(2, 2, n, H, W, cout)
    y = jnp.transpose(y, (2, 3, 0, 4, 1, 5)).reshape(n, 2 * H, 2 * W, cout)
    return y


def _pool_body(H, W, x_ref, o_ref):
    x = x_ref[0]
    m = x[0:H, 0:W, :]
    for dy in range(3):
        for dx in range(3):
            if dy == 0 and dx == 0:
                continue
            m = jnp.maximum(m, x[dy:dy + H, dx:dx + W, :])
    o_ref[0] = m


def _pool_body_cf(H, W, x_ref, o_ref):
    x = x_ref[0]
    m = x[:, 0:H, 0:W]
    for dy in range(3):
        for dx in range(3):
            if dy == 0 and dx == 0:
                continue
            m = jnp.maximum(m, x[:, dy:dy + H, dx:dx + W])
    o_ref[0] = m


def _maxpool3_nchw(x):
    n, c, H, W = x.shape
    xp = jnp.pad(x, ((0, 0), (0, 0), (1, 1), (1, 1)),
                 constant_values=-jnp.inf)
    return pl.pallas_call(
        functools.partial(_pool_body_cf, H, W),
        grid=(n,),
        in_specs=[pl.BlockSpec((1, c, H + 2, W + 2), lambda i: (i, 0, 0, 0))],
        out_specs=pl.BlockSpec((1, c, H, W), lambda i: (i, 0, 0, 0)),
        out_shape=jax.ShapeDtypeStruct((n, c, H, W), jnp.float32),
    )(xp)


def _maxpool3(x):
    n, H, W, c = x.shape
    xp = jnp.pad(x, ((0, 0), (1, 1), (1, 1), (0, 0)),
                 constant_values=-jnp.inf)
    return pl.pallas_call(
        functools.partial(_pool_body, H, W),
        grid=(n,),
        in_specs=[pl.BlockSpec((1, H + 2, W + 2, c), lambda i: (i, 0, 0, 0))],
        out_specs=pl.BlockSpec((1, H, W, c), lambda i: (i, 0, 0, 0)),
        out_shape=jax.ShapeDtypeStruct((n, H, W, c), jnp.float32),
    )(xp)


def kernel(x, E, W1, b1, W2, b2, W3, b3, W4, b4, W5, b5):
    n, cz, hz, wz = x.shape
    V, D = E.shape
    flat = jnp.transpose(x, (0, 2, 3, 1)).reshape(-1, D)

    idx2 = _vq_argmin(flat, E)
    q = _make_sc_gather(V, D, flat.shape[0])(E, idx2.reshape(-1))
    loss, ppx, qst = _stats(flat, q, idx2)

    h = qst.reshape(n, hz, wz, cz)
    for w, b in ((W1, b1), (W2, b2), (W3, b3), (W4, b4)):
        hp = jnp.pad(h, ((0, 0), (1, 1), (1, 1), (0, 0)))
        h = _maxpool3(_deconv_relu(hp, _prep_w(w), b))

    hp = jnp.pad(h, ((0, 0), (1, 1), (1, 1), (0, 0)))
    H = hp.shape[1] - 2
    Wd = hp.shape[2] - 2
    ph = _deconv_relu_phases(hp, _prep_w(W5), b5)
    y = ph.reshape(n, 2, 2, H, Wd, 3)
    y = jnp.transpose(y, (0, 5, 3, 1, 4, 2)).reshape(n, 3, 2 * H, 2 * Wd)
    h = _maxpool3_nchw(y)

    return (loss[0, 0], h, ppx[0, 0])


# speedup vs baseline: 1.1289x; 1.0903x over previous
import functools

import jax
import jax.numpy as jnp
from jax import lax
from jax.experimental import pallas as pl
from jax.experimental.pallas import tpu as pltpu
from jax.experimental.pallas import tpu_sc as plsc


def _vq_body(flat_ref, e_ref, idx_ref):
    flat = flat_ref[...]
    e = e_ref[...]
    fsq = jnp.sum(flat * flat, axis=1, keepdims=True)
    esq = jnp.sum(e * e, axis=1)
    m = lax.dot_general(flat, e, (((1,), (1,)), ((), ())),
                        preferred_element_type=jnp.float32)
    d = fsq + esq[None, :] - 2.0 * m
    mn = jnp.min(d, axis=1, keepdims=True)
    ks = lax.broadcasted_iota(jnp.int32, d.shape, 1)
    idx = jnp.min(jnp.where(d == mn, ks, jnp.int32(2 ** 30)), axis=1)
    idx_ref[...] = idx[:, None]


def _vq_argmin(flat, e):
    return pl.pallas_call(
        _vq_body,
        out_shape=jax.ShapeDtypeStruct((flat.shape[0], 1), jnp.int32),
    )(flat, e)


def _make_sc_gather(V, D, B):
    info = plsc.get_sparse_core_info()
    num_cores = info.num_cores
    b_per_w = B // (num_cores * info.num_subcores)
    mesh = plsc.VectorSubcoreMesh(core_axis_name="c", subcore_axis_name="s")

    @functools.partial(
        pl.kernel, mesh=mesh,
        out_type=jax.ShapeDtypeStruct((B, D), jnp.float32),
        scratch_types=[
            pltpu.VMEM((b_per_w,), jnp.int32),
            pltpu.VMEM((b_per_w, D), jnp.float32),
            pltpu.SemaphoreType.DMA,
        ],
    )
    def gather(table_hbm, idx_hbm, out_hbm, idx_v, rows_v, sem):
        wid = lax.axis_index("s") * num_cores + lax.axis_index("c")
        base = wid * b_per_w
        pltpu.sync_copy(idx_hbm.at[pl.ds(base, b_per_w)], idx_v)
        pltpu.async_copy(table_hbm.at[idx_v], rows_v, sem).wait()
        pltpu.sync_copy(rows_v, out_hbm.at[pl.ds(base, b_per_w)])

    return gather


def _stats_body(flat_ref, q_ref, idx_ref, loss_ref, ppx_ref, qst_ref):
    flat = flat_ref[...]
    q = q_ref[...]
    diff = q - flat
    v = jnp.mean(diff * diff)
    loss_ref[...] = jnp.reshape(v + 0.25 * v, (1, 1))
    n = flat.shape[0]
    idx = idx_ref[...]
    eq = (idx == idx.reshape(1, n)).astype(jnp.float32)
    avg = jnp.sum(eq, axis=1) * (1.0 / n)
    ent = jnp.mean(jnp.log(avg + 1e-10))
    ppx_ref[...] = jnp.reshape(jnp.exp(-ent), (1, 1))
    qst_ref[...] = flat + (q - flat)


def _stats(flat, q, idx2):
    n, d = flat.shape
    return pl.pallas_call(
        _stats_body,
        out_shape=[jax.ShapeDtypeStruct((1, 1), jnp.float32),
                   jax.ShapeDtypeStruct((1, 1), jnp.float32),
                   jax.ShapeDtypeStruct((n, d), jnp.float32)],
    )(flat, q, idx2)


def _prep_w(w):
    wt = jnp.transpose(w, (2, 3, 0, 1))
    rows = []
    for a in range(2):
        cols = []
        for b in range(2):
            t1 = []
            for dy in range(2):
                t2 = [wt[3 - a - 2 * dy, 3 - b - 2 * dx] for dx in range(2)]
                t1.append(jnp.stack(t2))
            cols.append(jnp.stack(t1))
        rows.append(jnp.stack(cols))
    return jnp.stack(rows)


def _conv_body(H, W, cin, cout, x_ref, w_ref, b_ref, o_ref):
    a = pl.program_id(1)
    b = pl.program_id(2)
    bias = b_ref[...]
    for aa in range(2):
        for bb in range(2):
            @pl.when(jnp.logical_and(a == aa, b == bb))
            def _(aa=aa, bb=bb):
                acc = None
                for dy in range(2):
                    for dx in range(2):
                        xw = x_ref[0, aa + dy:aa + dy + H,
                                   bb + dx:bb + dx + W, :]
                        p = jnp.dot(xw.reshape(H * W, cin),
                                    w_ref[0, 0, dy, dx],
                                    preferred_element_type=jnp.float32)
                        acc = p if acc is None else acc + p
                o_ref[0, 0, 0] = jnp.maximum(acc + bias, 0.0)


def _conv_body_nb(N, H, W, cin, cout, x_ref, w_ref, b_ref, o_ref):
    a = pl.program_id(0)
    b = pl.program_id(1)
    bias = b_ref[...]
    for aa in range(2):
        for bb in range(2):
            @pl.when(jnp.logical_and(a == aa, b == bb))
            def _(aa=aa, bb=bb):
                acc = None
                for dy in range(2):
                    for dx in range(2):
                        xw = x_ref[:, aa + dy:aa + dy + H,
                                   bb + dx:bb + dx + W, :]
                        p = jnp.dot(xw.reshape(N * H * W, cin),
                                    w_ref[0, 0, dy, dx],
                                    preferred_element_type=jnp.float32)
                        acc = p if acc is None else acc + p
                o_ref[0, 0] = jnp.maximum(acc + bias, 0.0)


def _conv_pool_body(H, W, cin, cout, x_ref, w_ref, b_ref, o_ref):
    bias = b_ref[...]
    ph = [[None, None], [None, None]]
    for a in range(2):
        for b in range(2):
            acc = None
            for dy in range(2):
                for dx in range(2):
                    xw = x_ref[0, a + dy:a + dy + H, b + dx:b + dx + W, :]
                    p = jnp.dot(xw.reshape(H * W, cin), w_ref[a, b, dy, dx],
                                preferred_element_type=jnp.float32)
                    acc = p if acc is None else acc + p
            ph[a][b] = jnp.maximum(acc + bias, 0.0).reshape(H, W, cout)

    def cm1(t):
        z = jnp.zeros_like(t[:, :1, :])
        return jnp.concatenate([z, t[:, :-1, :]], axis=1)

    def cp1(t):
        z = jnp.zeros_like(t[:, :1, :])
        return jnp.concatenate([t[:, 1:, :], z], axis=1)

    def rm1(t):
        z = jnp.zeros_like(t[:1])
        return jnp.concatenate([z, t[:-1]], axis=0)

    def rp1(t):
        z = jnp.zeros_like(t[:1])
        return jnp.concatenate([t[1:], z], axis=0)

    cp = [[None, None], [None, None]]
    for ap in range(2):
        cp[ap][0] = jnp.maximum(jnp.maximum(cm1(ph[ap][1]), ph[ap][0]),
                                ph[ap][1])
        cp[ap][1] = jnp.maximum(jnp.maximum(ph[ap][0], ph[ap][1]),
                                cp1(ph[ap][0]))
    for b in range(2):
        o_ref[0, 0, b] = jnp.maximum(jnp.maximum(rm1(cp[1][b]), cp[0][b]),
                                     cp[1][b])
        o_ref[0, 1, b] = jnp.maximum(jnp.maximum(cp[0][b], cp[1][b]),
                                     rp1(cp[0][b]))


def _deconv_relu_pool(x, wr, bias):
    n, h2, w2, cin = x.shape
    H, W = h2 - 2, w2 - 2
    cout = wr.shape[-1]
    out = pl.pallas_call(
        functools.partial(_conv_pool_body, H, W, cin, cout),
        grid=(n,),
        in_specs=[
            pl.BlockSpec((1, H + 2, W + 2, cin), lambda i: (i, 0, 0, 0)),
            pl.BlockSpec((2, 2, 2, 2, cin, cout),
                         lambda i: (0, 0, 0, 0, 0, 0)),
            pl.BlockSpec((1, cout), lambda i: (0, 0)),
        ],
        out_specs=pl.BlockSpec((1, 2, 2, H, W, cout),
                               lambda i: (i, 0, 0, 0, 0, 0)),
        out_shape=jax.ShapeDtypeStruct((n, 2, 2, H, W, cout), jnp.float32),
    )(x, wr, bias.reshape(1, cout))
    y = jnp.transpose(out, (0, 3, 1, 4, 2, 5)).reshape(n, 2 * H, 2 * W, cout)
    return y


def _deconv_relu_phases_nb(x, wr, bias):
    n, h2, w2, cin = x.shape
    H, W = h2 - 2, w2 - 2
    cout = wr.shape[-1]
    return pl.pallas_call(
        functools.partial(_conv_body_nb, n, H, W, cin, cout),
        grid=(2, 2),
        in_specs=[
            pl.BlockSpec((n, H + 2, W + 2, cin), lambda a, b: (0, 0, 0, 0)),
            pl.BlockSpec((1, 1, 2, 2, cin, cout),
                         lambda a, b: (a, b, 0, 0, 0, 0)),
            pl.BlockSpec((1, cout), lambda a, b: (0, 0)),
        ],
        out_specs=pl.BlockSpec((1, 1, n * H * W, cout),
                               lambda a, b: (a, b, 0, 0)),
        out_shape=jax.ShapeDtypeStruct((2, 2, n * H * W, cout), jnp.float32),
    )(x, wr, bias.reshape(1, cout))


def _deconv_relu_phases(x, wr, bias):
    n, h2, w2, cin = x.shape
    H, W = h2 - 2, w2 - 2
    cout = wr.shape[-1]
    return pl.pallas_call(
        functools.partial(_conv_body, H, W, cin, cout),
        grid=(n, 2, 2),
        in_specs=[
            pl.BlockSpec((1, H + 2, W + 2, cin), lambda i, a, b: (i, 0, 0, 0)),
            pl.BlockSpec((1, 1, 2, 2, cin, cout),
                         lambda i, a, b: (a, b, 0, 0, 0, 0)),
            pl.BlockSpec((1, cout), lambda i, a, b: (0, 0)),
        ],
        out_specs=pl.BlockSpec((1, 1, 1, H * W, cout),
                               lambda i, a, b: (i, a, b, 0, 0)),
        out_shape=jax.ShapeDtypeStruct((n, 2, 2, H * W, cout), jnp.float32),
    )(x, wr, bias.reshape(1, cout))


def _deconv_relu(x, wr, bias):
    n, h2, w2, cin = x.shape
    H, W = h2 - 2, w2 - 2
    cout = wr.shape[-1]
    out = _deconv_relu_phases_nb(x, wr, bias)
    y = out.reshape(2, 2, n, H, W, cout)
    y = jnp.transpose(y, (2, 3, 0, 4, 1, 5)).reshape(n, 2 * H, 2 * W, cout)
    return y


def _pool_body(H, W, x_ref, o_ref):
    x = x_ref[0]
    m = x[0:H, 0:W, :]
    for dy in range(3):
        for dx in range(3):
            if dy == 0 and dx == 0:
                continue
            m = jnp.maximum(m, x[dy:dy + H, dx:dx + W, :])
    o_ref[0] = m


def _pool_body_cf(H, W, x_ref, o_ref):
    x = x_ref[0]
    m = x[:, 0:H, 0:W]
    for dy in range(3):
        for dx in range(3):
            if dy == 0 and dx == 0:
                continue
            m = jnp.maximum(m, x[:, dy:dy + H, dx:dx + W])
    o_ref[0] = m


def _maxpool3_nchw(x):
    n, c, H, W = x.shape
    xp = jnp.pad(x, ((0, 0), (0, 0), (1, 1), (1, 1)),
                 constant_values=-jnp.inf)
    return pl.pallas_call(
        functools.partial(_pool_body_cf, H, W),
        grid=(n,),
        in_specs=[pl.BlockSpec((1, c, H + 2, W + 2), lambda i: (i, 0, 0, 0))],
        out_specs=pl.BlockSpec((1, c, H, W), lambda i: (i, 0, 0, 0)),
        out_shape=jax.ShapeDtypeStruct((n, c, H, W), jnp.float32),
    )(xp)


def _maxpool3(x):
    n, H, W, c = x.shape
    xp = jnp.pad(x, ((0, 0), (1, 1), (1, 1), (0, 0)),
                 constant_values=-jnp.inf)
    return pl.pallas_call(
        functools.partial(_pool_body, H, W),
        grid=(n,),
        in_specs=[pl.BlockSpec((1, H + 2, W + 2, c), lambda i: (i, 0, 0, 0))],
        out_specs=pl.BlockSpec((1, H, W, c), lambda i: (i, 0, 0, 0)),
        out_shape=jax.ShapeDtypeStruct((n, H, W, c), jnp.float32),
    )(xp)


def kernel(x, E, W1, b1, W2, b2, W3, b3, W4, b4, W5, b5):
    n, cz, hz, wz = x.shape
    V, D = E.shape
    flat = jnp.transpose(x, (0, 2, 3, 1)).reshape(-1, D)

    idx2 = _vq_argmin(flat, E)
    q = _make_sc_gather(V, D, flat.shape[0])(E, idx2.reshape(-1))
    loss, ppx, qst = _stats(flat, q, idx2)

    h = qst.reshape(n, hz, wz, cz)
    for w, b in ((W1, b1), (W2, b2), (W3, b3), (W4, b4)):
        hp = jnp.pad(h, ((0, 0), (1, 1), (1, 1), (0, 0)))
        h = _deconv_relu_pool(hp, _prep_w(w), b)

    hp = jnp.pad(h, ((0, 0), (1, 1), (1, 1), (0, 0)))
    H = hp.shape[1] - 2
    Wd = hp.shape[2] - 2
    ph = _deconv_relu_phases(hp, _prep_w(W5), b5)
    y = ph.reshape(n, 2, 2, H, Wd, 3)
    y = jnp.transpose(y, (0, 5, 3, 1, 4, 2)).reshape(n, 3, 2 * H, 2 * Wd)
    h = _maxpool3_nchw(y)

    return (loss[0, 0], h, ppx[0, 0])


# speedup vs baseline: 2.1943x; 1.9438x over previous
import functools

import jax
import jax.numpy as jnp
from jax import lax
from jax.experimental import pallas as pl
from jax.experimental.pallas import tpu as pltpu
from jax.experimental.pallas import tpu_sc as plsc


def _vq_body(flat_ref, e_ref, idx_ref):
    flat = flat_ref[...]
    e = e_ref[...]
    fsq = jnp.sum(flat * flat, axis=1, keepdims=True)
    esq = jnp.sum(e * e, axis=1)
    m = lax.dot_general(flat, e, (((1,), (1,)), ((), ())),
                        preferred_element_type=jnp.float32)
    d = fsq + esq[None, :] - 2.0 * m
    mn = jnp.min(d, axis=1, keepdims=True)
    ks = lax.broadcasted_iota(jnp.int32, d.shape, 1)
    idx = jnp.min(jnp.where(d == mn, ks, jnp.int32(2 ** 30)), axis=1)
    idx_ref[...] = idx[:, None]


def _vq_argmin(flat, e):
    return pl.pallas_call(
        _vq_body,
        out_shape=jax.ShapeDtypeStruct((flat.shape[0], 1), jnp.int32),
    )(flat, e)


def _make_sc_gather(V, D, B):
    info = plsc.get_sparse_core_info()
    num_cores = info.num_cores
    b_per_w = B // (num_cores * info.num_subcores)
    mesh = plsc.VectorSubcoreMesh(core_axis_name="c", subcore_axis_name="s")

    @functools.partial(
        pl.kernel, mesh=mesh,
        out_type=jax.ShapeDtypeStruct((B, D), jnp.float32),
        scratch_types=[
            pltpu.VMEM((b_per_w,), jnp.int32),
            pltpu.VMEM((b_per_w, D), jnp.float32),
            pltpu.SemaphoreType.DMA,
        ],
    )
    def gather(table_hbm, idx_hbm, out_hbm, idx_v, rows_v, sem):
        wid = lax.axis_index("s") * num_cores + lax.axis_index("c")
        base = wid * b_per_w
        pltpu.sync_copy(idx_hbm.at[pl.ds(base, b_per_w)], idx_v)
        pltpu.async_copy(table_hbm.at[idx_v], rows_v, sem).wait()
        pltpu.sync_copy(rows_v, out_hbm.at[pl.ds(base, b_per_w)])

    return gather


def _stats_body(flat_ref, q_ref, idx_ref, loss_ref, ppx_ref, qst_ref):
    flat = flat_ref[...]
    q = q_ref[...]
    diff = q - flat
    v = jnp.mean(diff * diff)
    loss_ref[...] = jnp.reshape(v + 0.25 * v, (1, 1))
    n = flat.shape[0]
    idx = idx_ref[...]
    eq = (idx == idx.reshape(1, n)).astype(jnp.float32)
    avg = jnp.sum(eq, axis=1) * (1.0 / n)
    ent = jnp.mean(jnp.log(avg + 1e-10))
    ppx_ref[...] = jnp.reshape(jnp.exp(-ent), (1, 1))
    qst_ref[...] = flat + (q - flat)


def _stats(flat, q, idx2):
    n, d = flat.shape
    return pl.pallas_call(
        _stats_body,
        out_shape=[jax.ShapeDtypeStruct((1, 1), jnp.float32),
                   jax.ShapeDtypeStruct((1, 1), jnp.float32),
                   jax.ShapeDtypeStruct((n, d), jnp.float32)],
    )(flat, q, idx2)


def _prep_w(w):
    wt = jnp.transpose(w, (2, 3, 0, 1))
    rows = []
    for a in range(2):
        cols = []
        for b in range(2):
            t1 = []
            for dy in range(2):
                t2 = [wt[3 - a - 2 * dy, 3 - b - 2 * dx] for dx in range(2)]
                t1.append(jnp.stack(t2))
            cols.append(jnp.stack(t1))
        rows.append(jnp.stack(cols))
    return jnp.stack(rows)


def _conv_body(H, W, cin, cout, x_ref, w_ref, b_ref, o_ref):
    a = pl.program_id(1)
    b = pl.program_id(2)
    bias = b_ref[...]
    for aa in range(2):
        for bb in range(2):
            @pl.when(jnp.logical_and(a == aa, b == bb))
            def _(aa=aa, bb=bb):
                acc = None
                for dy in range(2):
                    for dx in range(2):
                        xw = x_ref[0, aa + dy:aa + dy + H,
                                   bb + dx:bb + dx + W, :]
                        p = jnp.dot(xw.reshape(H * W, cin),
                                    w_ref[0, 0, dy, dx],
                                    preferred_element_type=jnp.float32)
                        acc = p if acc is None else acc + p
                o_ref[0, 0, 0] = jnp.maximum(acc + bias, 0.0)


def _conv_body_nb(N, H, W, cin, cout, x_ref, w_ref, b_ref, o_ref):
    a = pl.program_id(0)
    b = pl.program_id(1)
    bias = b_ref[...]
    for aa in range(2):
        for bb in range(2):
            @pl.when(jnp.logical_and(a == aa, b == bb))
            def _(aa=aa, bb=bb):
                acc = None
                for dy in range(2):
                    for dx in range(2):
                        xw = x_ref[:, aa + dy:aa + dy + H,
                                   bb + dx:bb + dx + W, :]
                        p = jnp.dot(xw.reshape(N * H * W, cin),
                                    w_ref[0, 0, dy, dx],
                                    preferred_element_type=jnp.float32)
                        acc = p if acc is None else acc + p
                o_ref[0, 0] = jnp.maximum(acc + bias, 0.0)


def _conv_pool_body(H, W, cin, cout, x_ref, w_ref, b_ref, o_ref):
    bias = b_ref[...]
    ph = [[None, None], [None, None]]
    for a in range(2):
        for b in range(2):
            acc = None
            for dy in range(2):
                for dx in range(2):
                    xw = x_ref[0, a + dy:a + dy + H, b + dx:b + dx + W, :]
                    p = jnp.dot(xw.reshape(H * W, cin), w_ref[a, b, dy, dx],
                                preferred_element_type=jnp.float32)
                    acc = p if acc is None else acc + p
            ph[a][b] = jnp.maximum(acc + bias, 0.0).reshape(H, W, cout)

    def cm1(t):
        z = jnp.zeros_like(t[:, :1, :])
        return jnp.concatenate([z, t[:, :-1, :]], axis=1)

    def cp1(t):
        z = jnp.zeros_like(t[:, :1, :])
        return jnp.concatenate([t[:, 1:, :], z], axis=1)

    def rm1(t):
        z = jnp.zeros_like(t[:1])
        return jnp.concatenate([z, t[:-1]], axis=0)

    def rp1(t):
        z = jnp.zeros_like(t[:1])
        return jnp.concatenate([t[1:], z], axis=0)

    cp = [[None, None], [None, None]]
    for ap in range(2):
        cp[ap][0] = jnp.maximum(jnp.maximum(cm1(ph[ap][1]), ph[ap][0]),
                                ph[ap][1])
        cp[ap][1] = jnp.maximum(jnp.maximum(ph[ap][0], ph[ap][1]),
                                cp1(ph[ap][0]))
    for b in range(2):
        o_ref[0, 0, b] = jnp.maximum(jnp.maximum(rm1(cp[1][b]), cp[0][b]),
                                     cp[1][b])
        o_ref[0, 1, b] = jnp.maximum(jnp.maximum(cp[0][b], cp[1][b]),
                                     rp1(cp[0][b]))


def _deconv_relu_pool(x, wr, bias):
    n, h2, w2, cin = x.shape
    H, W = h2 - 2, w2 - 2
    cout = wr.shape[-1]
    out = pl.pallas_call(
        functools.partial(_conv_pool_body, H, W, cin, cout),
        grid=(n,),
        in_specs=[
            pl.BlockSpec((1, H + 2, W + 2, cin), lambda i: (i, 0, 0, 0)),
            pl.BlockSpec((2, 2, 2, 2, cin, cout),
                         lambda i: (0, 0, 0, 0, 0, 0)),
            pl.BlockSpec((1, cout), lambda i: (0, 0)),
        ],
        out_specs=pl.BlockSpec((1, 2, 2, H, W, cout),
                               lambda i: (i, 0, 0, 0, 0, 0)),
        out_shape=jax.ShapeDtypeStruct((n, 2, 2, H, W, cout), jnp.float32),
    )(x, wr, bias.reshape(1, cout))
    y = jnp.transpose(out, (0, 3, 1, 4, 2, 5)).reshape(n, 2 * H, 2 * W, cout)
    return y


def _deconv_relu_phases_nb(x, wr, bias):
    n, h2, w2, cin = x.shape
    H, W = h2 - 2, w2 - 2
    cout = wr.shape[-1]
    return pl.pallas_call(
        functools.partial(_conv_body_nb, n, H, W, cin, cout),
        grid=(2, 2),
        in_specs=[
            pl.BlockSpec((n, H + 2, W + 2, cin), lambda a, b: (0, 0, 0, 0)),
            pl.BlockSpec((1, 1, 2, 2, cin, cout),
                         lambda a, b: (a, b, 0, 0, 0, 0)),
            pl.BlockSpec((1, cout), lambda a, b: (0, 0)),
        ],
        out_specs=pl.BlockSpec((1, 1, n * H * W, cout),
                               lambda a, b: (a, b, 0, 0)),
        out_shape=jax.ShapeDtypeStruct((2, 2, n * H * W, cout), jnp.float32),
    )(x, wr, bias.reshape(1, cout))


def _deconv_relu_phases(x, wr, bias):
    n, h2, w2, cin = x.shape
    H, W = h2 - 2, w2 - 2
    cout = wr.shape[-1]
    return pl.pallas_call(
        functools.partial(_conv_body, H, W, cin, cout),
        grid=(n, 2, 2),
        in_specs=[
            pl.BlockSpec((1, H + 2, W + 2, cin), lambda i, a, b: (i, 0, 0, 0)),
            pl.BlockSpec((1, 1, 2, 2, cin, cout),
                         lambda i, a, b: (a, b, 0, 0, 0, 0)),
            pl.BlockSpec((1, cout), lambda i, a, b: (0, 0)),
        ],
        out_specs=pl.BlockSpec((1, 1, 1, H * W, cout),
                               lambda i, a, b: (i, a, b, 0, 0)),
        out_shape=jax.ShapeDtypeStruct((n, 2, 2, H * W, cout), jnp.float32),
    )(x, wr, bias.reshape(1, cout))


def _deconv_relu(x, wr, bias):
    n, h2, w2, cin = x.shape
    H, W = h2 - 2, w2 - 2
    cout = wr.shape[-1]
    out = _deconv_relu_phases_nb(x, wr, bias)
    y = out.reshape(2, 2, n, H, W, cout)
    y = jnp.transpose(y, (2, 3, 0, 4, 1, 5)).reshape(n, 2 * H, 2 * W, cout)
    return y


def _pool_body(H, W, x_ref, o_ref):
    x = x_ref[0]
    m = x[0:H, 0:W, :]
    for dy in range(3):
        for dx in range(3):
            if dy == 0 and dx == 0:
                continue
            m = jnp.maximum(m, x[dy:dy + H, dx:dx + W, :])
    o_ref[0] = m


def _pool_body_cf(H, W, x_ref, o_ref):
    x = x_ref[0]
    m = x[:, 0:H, 0:W]
    for dy in range(3):
        for dx in range(3):
            if dy == 0 and dx == 0:
                continue
            m = jnp.maximum(m, x[:, dy:dy + H, dx:dx + W])
    o_ref[0] = m


def _maxpool3_nchw(x):
    n, c, H, W = x.shape
    xp = jnp.pad(x, ((0, 0), (0, 0), (1, 1), (1, 1)),
                 constant_values=-jnp.inf)
    return pl.pallas_call(
        functools.partial(_pool_body_cf, H, W),
        grid=(n,),
        in_specs=[pl.BlockSpec((1, c, H + 2, W + 2), lambda i: (i, 0, 0, 0))],
        out_specs=pl.BlockSpec((1, c, H, W), lambda i: (i, 0, 0, 0)),
        out_shape=jax.ShapeDtypeStruct((n, c, H, W), jnp.float32),
    )(xp)


def _maxpool3(x):
    n, H, W, c = x.shape
    xp = jnp.pad(x, ((0, 0), (1, 1), (1, 1), (0, 0)),
                 constant_values=-jnp.inf)
    return pl.pallas_call(
        functools.partial(_pool_body, H, W),
        grid=(n,),
        in_specs=[pl.BlockSpec((1, H + 2, W + 2, c), lambda i: (i, 0, 0, 0))],
        out_specs=pl.BlockSpec((1, H, W, c), lambda i: (i, 0, 0, 0)),
        out_shape=jax.ShapeDtypeStruct((n, H, W, c), jnp.float32),
    )(xp)


def _prep_w5(w):
    cin, cout = w.shape[0], w.shape[1]
    blocks = []
    for u in range(3):
        for v in range(3):
            cols = []
            for a in range(2):
                for b in range(2):
                    dy, dx = u - a, v - b
                    if 0 <= dy <= 1 and 0 <= dx <= 1:
                        cols.append(w[:, :, 3 + a - 2 * u, 3 + b - 2 * v])
                    else:
                        cols.append(jnp.zeros((cin, cout), w.dtype))
            blocks.append(jnp.concatenate(cols, axis=1))
    return jnp.concatenate(blocks, axis=0)


def _conv5_pool_body(H, W, cin, cout, x_ref, w_ref, b_ref, o_ref):
    wins = []
    for u in range(3):
        for v in range(3):
            wins.append(x_ref[0, u:u + H, v:v + W, :].reshape(H * W, cin))
    lhs = jnp.concatenate(wins, axis=1)
    y = jnp.dot(lhs, w_ref[...], preferred_element_type=jnp.float32)
    y = jnp.maximum(y + b_ref[...], 0.0)
    ph = [[None, None], [None, None]]
    for a in range(2):
        for b in range(2):
            c0 = (a * 2 + b) * cout
            t = jnp.transpose(y[:, c0:c0 + cout])
            ph[a][b] = t.reshape(cout, H, W)

    def cm1(t):
        z = jnp.zeros_like(t[:, :, :1])
        return jnp.concatenate([z, t[:, :, :-1]], axis=2)

    def cp1(t):
        z = jnp.zeros_like(t[:, :, :1])
        return jnp.concatenate([t[:, :, 1:], z], axis=2)

    def rm1(t):
        z = jnp.zeros_like(t[:, :1])
        return jnp.concatenate([z, t[:, :-1]], axis=1)

    def rp1(t):
        z = jnp.zeros_like(t[:, :1])
        return jnp.concatenate([t[:, 1:], z], axis=1)

    cp = [[None, None], [None, None]]
    for ap in range(2):
        cp[ap][0] = jnp.maximum(jnp.maximum(cm1(ph[ap][1]), ph[ap][0]),
                                ph[ap][1])
        cp[ap][1] = jnp.maximum(jnp.maximum(ph[ap][0], ph[ap][1]),
                                cp1(ph[ap][0]))
    for b in range(2):
        o_ref[0, 0, b] = jnp.maximum(jnp.maximum(rm1(cp[1][b]), cp[0][b]),
                                     cp[1][b])
        o_ref[0, 1, b] = jnp.maximum(jnp.maximum(cp[0][b], cp[1][b]),
                                     rp1(cp[0][b]))


def _deconv_relu_pool_last(x, w, bias):
    n, h2, w2, cin = x.shape
    H, W = h2 - 2, w2 - 2
    cout = w.shape[1]
    wr = _prep_w5(w)
    out = pl.pallas_call(
        functools.partial(_conv5_pool_body, H, W, cin, cout),
        grid=(n,),
        in_specs=[
            pl.BlockSpec((1, H + 2, W + 2, cin), lambda i: (i, 0, 0, 0)),
            pl.BlockSpec((9 * cin, 4 * cout), lambda i: (0, 0)),
            pl.BlockSpec((1, 4 * cout), lambda i: (0, 0)),
        ],
        out_specs=pl.BlockSpec((1, 2, 2, cout, H, W),
                               lambda i: (i, 0, 0, 0, 0, 0)),
        out_shape=jax.ShapeDtypeStruct((n, 2, 2, cout, H, W), jnp.float32),
    )(x, wr, jnp.tile(bias, 4).reshape(1, 4 * cout))
    y = jnp.transpose(out, (0, 3, 4, 1, 5, 2)).reshape(n, cout, 2 * H, 2 * W)
    return y


def kernel(x, E, W1, b1, W2, b2, W3, b3, W4, b4, W5, b5):
    n, cz, hz, wz = x.shape
    V, D = E.shape
    flat = jnp.transpose(x, (0, 2, 3, 1)).reshape(-1, D)

    idx2 = _vq_argmin(flat, E)
    q = _make_sc_gather(V, D, flat.shape[0])(E, idx2.reshape(-1))
    loss, ppx, qst = _stats(flat, q, idx2)

    h = qst.reshape(n, hz, wz, cz)
    for w, b in ((W1, b1), (W2, b2), (W3, b3), (W4, b4)):
        hp = jnp.pad(h, ((0, 0), (1, 1), (1, 1), (0, 0)))
        h = _deconv_relu_pool(hp, _prep_w(w), b)

    hp = jnp.pad(h, ((0, 0), (1, 1), (1, 1), (0, 0)))
    h = _deconv_relu_pool_last(hp, W5, b5)

    return (loss[0, 0], h, ppx[0, 0])
